# baseline (speedup 1.0000x reference)
import functools

import jax
import jax.numpy as jnp
from jax import lax
from jax.experimental import pallas as pl
from jax.experimental.pallas import tpu as pltpu
from jax.experimental.pallas import tpu_sc as plsc

_VOCAB = 100000
_D = 1024
_B = 4
_S = 2048
_N = _B * _S
_EPS = 1e-5
_L = 16
_DV = _D // _L

_NC = 2
_NS = 16
_NW = _NC * _NS
_PER_W = _N // _NW
_CHUNK = 32
_NCHUNK = _PER_W // _CHUNK
_NBUF = 3


def _rsqrt_newton(a):
    y = jnp.full((_L,), 0.7 * 2.0 ** (-3), jnp.float32)
    for k in range(2, -10, -1):
        thresh = jnp.float32(4.0**k)
        guess = jnp.float32(0.7 * 2.0 ** (-k))
        y = jnp.where(a < thresh, guess, y)
    for _ in range(7):
        y = y * (1.5 - (0.5 * a) * y * y)
    return y


def _sc_gather_ln(ids, table, gamma, beta):
    mesh = plsc.VectorSubcoreMesh(core_axis_name="c", subcore_axis_name="s")

    @functools.partial(
        pl.kernel,
        mesh=mesh,
        out_type=jax.ShapeDtypeStruct((_N, _D), jnp.float32),
        scratch_types=[
            pltpu.VMEM((_PER_W,), jnp.int32),
            pltpu.VMEM((_D,), jnp.float32),
            pltpu.VMEM((_D,), jnp.float32),
        ]
        + [pltpu.VMEM((_CHUNK, _D), jnp.float32)] * _NBUF
        + [pltpu.SemaphoreType.DMA] * (2 * _NBUF + 1),
    )
    def fused_kernel(ids_hbm, table_hbm, gamma_hbm, beta_hbm, out_hbm, *scratch):
        idx_v, gv, bv = scratch[0], scratch[1], scratch[2]
        bufs = scratch[3 : 3 + _NBUF]
        gsems = scratch[3 + _NBUF : 3 + 2 * _NBUF]
        wsems = scratch[3 + 2 * _NBUF : 3 + 3 * _NBUF]
        gbsem = scratch[3 + 3 * _NBUF]
        wid = lax.axis_index("s") * _NC + lax.axis_index("c")
        base = wid * _PER_W
        gb_copy = pltpu.async_copy(gamma_hbm, gv, gbsem)
        pltpu.sync_copy(ids_hbm.at[pl.ds(base, _PER_W)], idx_v)

        def start_gather(c):
            b = c % _NBUF
            return pltpu.async_copy(
                table_hbm.at[idx_v.at[pl.ds(c * _CHUNK, _CHUNK)]], bufs[b], gsems[b]
            )

        gcopy = [None] * _NCHUNK
        wcopy = [None] * _NCHUNK
        gcopy[0] = start_gather(0)
        gb_copy.wait()
        pltpu.sync_copy(beta_hbm, bv)

        inv_d = jnp.float32(1.0 / _D)
        zeros = jnp.zeros((_L,), jnp.float32)
        lanes = lax.iota(jnp.int32, _L)
        shuffles = [lanes ^ sh for sh in (8, 4, 2, 1)]

        def splat_sum(v):
            for idx in shuffles:
                v = v + v[idx]
            return v

        def normalize_chunk(buf):
            def row_body(r, _):
                def p1(j, carry):
                    s, s2 = carry
                    v = buf[r, pl.ds(j * _L, _L)]
                    return s + v, s2 + v * v

                s, s2 = lax.fori_loop(0, _DV, p1, (zeros, zeros), unroll=4)
                mu = splat_sum(s) * inv_d
                var = splat_sum(s2) * inv_d - mu * mu
                rs = _rsqrt_newton(var + _EPS)
                a_v = rs
                c_v = -mu * rs

                def p2(j, _):
                    col = pl.ds(j * _L, _L)
                    v = buf[r, col]
                    y = (v * a_v + c_v) * gv[col] + bv[col]
                    buf[r, col] = y
                    return 0

                lax.fori_loop(0, _DV, p2, 0, unroll=4)
                return 0

            lax.fori_loop(0, _CHUNK, row_body, 0)

        for c in range(_NCHUNK):
            b = c % _NBUF
            if c + 1 < _NCHUNK:
                if c + 1 >= _NBUF:
                    wcopy[c + 1 - _NBUF].wait()
                gcopy[c + 1] = start_gather(c + 1)
            gcopy[c].wait()
            normalize_chunk(bufs[b])
            wcopy[c] = pltpu.async_copy(
                bufs[b], out_hbm.at[pl.ds(base + c * _CHUNK, _CHUNK)], wsems[b]
            )
        for c in range(max(0, _NCHUNK - _NBUF), _NCHUNK):
            wcopy[c].wait()

    return fused_kernel(ids, table, gamma, beta)


def kernel(tp_inputs, table, gamma, beta):
    ids = tp_inputs[..., 0].reshape(_N)
    mask = tp_inputs[..., 1].astype(jnp.float32)
    hidden = _sc_gather_ln(ids, table, gamma, beta)
    return hidden.reshape(_B, _S, _D), mask


# speedup vs baseline: 2.7332x; 2.7332x over previous
import functools

import jax
import jax.numpy as jnp
from jax import lax
from jax.experimental import pallas as pl
from jax.experimental.pallas import tpu as pltpu
from jax.experimental.pallas import tpu_sc as plsc

_VOCAB = 100000
_D = 1024
_B = 4
_S = 2048
_N = _B * _S
_EPS = 1e-5

_NC = 2
_NS = 16
_NW = _NC * _NS
_NSLICE = 1
_SLICE = _N // _NSLICE
_PER_W = _SLICE // _NW
_CHUNK = 32
_NCHUNK = _PER_W // _CHUNK
_NBUF = 3


def _sc_gather_slice(ids, table, slice_idx):
    mesh = plsc.VectorSubcoreMesh(core_axis_name="c", subcore_axis_name="s")
    slice_off = slice_idx * _SLICE

    @functools.partial(
        pl.kernel,
        mesh=mesh,
        out_type=jax.ShapeDtypeStruct((_SLICE, _D), jnp.float32),
        scratch_types=[pltpu.VMEM((_PER_W,), jnp.int32)]
        + [pltpu.VMEM((_CHUNK, _D), jnp.float32)] * _NBUF
        + [pltpu.SemaphoreType.DMA] * (2 * _NBUF),
    )
    def gather_kernel(ids_hbm, table_hbm, out_hbm, idx_v, *scratch):
        bufs = scratch[:_NBUF]
        gsems = scratch[_NBUF : 2 * _NBUF]
        wsems = scratch[2 * _NBUF :]
        wid = lax.axis_index("s") * _NC + lax.axis_index("c")
        base = wid * _PER_W
        pltpu.sync_copy(ids_hbm.at[pl.ds(slice_off + base, _PER_W)], idx_v)

        def start_gather(c):
            b = c % _NBUF
            return pltpu.async_copy(
                table_hbm.at[idx_v.at[pl.ds(c * _CHUNK, _CHUNK)]], bufs[b], gsems[b]
            )

        gcopy = [None] * _NCHUNK
        wcopy = [None] * _NCHUNK
        gcopy[0] = start_gather(0)
        for c in range(_NCHUNK):
            b = c % _NBUF
            if c + 1 < _NCHUNK:
                if c + 1 >= _NBUF:
                    wcopy[c + 1 - _NBUF].wait()
                gcopy[c + 1] = start_gather(c + 1)
            gcopy[c].wait()
            wcopy[c] = pltpu.async_copy(
                bufs[b], out_hbm.at[pl.ds(base + c * _CHUNK, _CHUNK)], wsems[b]
            )
        for c in range(max(0, _NCHUNK - _NBUF), _NCHUNK):
            wcopy[c].wait()

    return gather_kernel(ids, table)


_LN_BLK = 2048
_BLK_PER_SLICE = _SLICE // _LN_BLK


def _ln_body_first(x_ref, g_ref, b_ref, o_ref):
    x = x_ref[...]
    mu = jnp.mean(x, axis=-1, keepdims=True)
    xc = x - mu
    var = jnp.mean(xc * xc, axis=-1, keepdims=True)
    o_ref[...] = xc * lax.rsqrt(var + _EPS) * g_ref[...] + b_ref[...]


def _ln_body_chain(x_ref, g_ref, b_ref, prev_ref, o_ref):
    del prev_ref
    _ln_body_first(x_ref, g_ref, b_ref, o_ref)


def _tc_layernorm_slice(x, gamma2d, beta2d, prev, slice_idx):
    off = slice_idx * _BLK_PER_SLICE
    out_spec = pl.BlockSpec((_LN_BLK, _D), lambda i: (i + off, 0))
    in_specs = [
        pl.BlockSpec((_LN_BLK, _D), lambda i: (i, 0)),
        pl.BlockSpec((1, _D), lambda i: (0, 0)),
        pl.BlockSpec((1, _D), lambda i: (0, 0)),
    ]
    out_shape = jax.ShapeDtypeStruct((_N, _D), jnp.float32)
    if prev is None:
        return pl.pallas_call(
            _ln_body_first,
            grid=(_BLK_PER_SLICE,),
            in_specs=in_specs,
            out_specs=out_spec,
            out_shape=out_shape,
        )(x, gamma2d, beta2d)
    return pl.pallas_call(
        _ln_body_chain,
        grid=(_BLK_PER_SLICE,),
        in_specs=in_specs + [pl.BlockSpec(memory_space=pltpu.HBM)],
        out_specs=out_spec,
        out_shape=out_shape,
        input_output_aliases={3: 0},
    )(x, gamma2d, beta2d, prev)


def kernel(tp_inputs, table, gamma, beta):
    ids = tp_inputs[..., 0].reshape(_N)
    mask = tp_inputs[..., 1].astype(jnp.float32)
    gamma2d = gamma.reshape(1, _D)
    beta2d = beta.reshape(1, _D)
    rows = [_sc_gather_slice(ids, table, k) for k in range(_NSLICE)]
    hidden = None
    for k in range(_NSLICE):
        hidden = _tc_layernorm_slice(rows[k], gamma2d, beta2d, hidden, k)
    return hidden.reshape(_B, _S, _D), mask
